# Initial kernel scaffold; baseline (speedup 1.0000x reference)
import functools

import jax
import jax.numpy as jnp
import numpy as np
from jax import lax
from jax.experimental import pallas as pl
from jax.experimental.pallas import tpu as pltpu
from jax.experimental.pallas import tpu_sc as plsc

_BATCH = 16384
_MAX_LEN = 200
_MASK_RATIO = 0.2

_NUM_WORKERS = 32
_ROWS_PER_GROUP = 16
_GROUPS = _BATCH // _ROWS_PER_GROUP
_GROUPS_PER_WORKER = _GROUPS // _NUM_WORKERS
_NUM_CORES = 2


def _score_perm_grouped() -> np.ndarray:
    scores = jax.random.uniform(jax.random.key(42), (_BATCH, _MAX_LEN),
                                dtype=jnp.float32)
    perm = np.asarray(jnp.argsort(scores, axis=1), dtype=np.int32)
    return np.ascontiguousarray(
        perm.reshape(_GROUPS, _ROWS_PER_GROUP, _MAX_LEN).transpose(0, 2, 1))


_PERM_G = _score_perm_grouped()


def _sc_body(item_hbm, len_hbm, perm_hbm, out_hbm, item_v, perm_v, len_v):
    wid = lax.axis_index("s") * _NUM_CORES + lax.axis_index("c")
    iota = lax.iota(jnp.int32, 16)
    zeros = jnp.zeros((16,), jnp.int32)

    def group_body(k, carry):
        g = wid * _GROUPS_PER_WORKER + k
        base = g * _ROWS_PER_GROUP
        pltpu.sync_copy(item_hbm.at[pl.ds(base, _ROWS_PER_GROUP), :], item_v)
        pltpu.sync_copy(perm_hbm.at[g], perm_v)
        pltpu.sync_copy(len_hbm.at[pl.ds(base, _ROWS_PER_GROUP)], len_v)
        lens = len_v[...]

        def p1(j, nv):
            col = plsc.load_gather(item_v, [iota, jnp.full((16,), j, jnp.int32)])
            v = (jnp.full((16,), j, jnp.int32) < lens) & (col != 0)
            return nv + v.astype(jnp.int32)

        nv = lax.fori_loop(0, _MAX_LEN, p1, zeros)
        m = (nv.astype(jnp.float32) * _MASK_RATIO).astype(jnp.int32)
        m = jnp.minimum(jnp.maximum(m, 1), nv)
        m = jnp.where((lens > 1) & (nv > 0), m, zeros)

        def p2(t, cnt):
            pcol = perm_v[t, :]
            itm = plsc.load_gather(item_v, [iota, pcol])
            v = (pcol < lens) & (itm != 0)
            sel = v & (cnt < m)
            plsc.store_scatter(item_v, [iota, pcol], zeros, mask=sel)
            return cnt + v.astype(jnp.int32)

        lax.fori_loop(0, _MAX_LEN, p2, zeros)
        pltpu.sync_copy(item_v, out_hbm.at[pl.ds(base, _ROWS_PER_GROUP), :])
        return carry

    lax.fori_loop(0, _GROUPS_PER_WORKER, group_body, 0)


_sc_kernel = functools.partial(
    pl.kernel,
    _sc_body,
    out_type=jax.ShapeDtypeStruct((_BATCH, _MAX_LEN), jnp.int32),
    mesh=plsc.VectorSubcoreMesh(core_axis_name="c", subcore_axis_name="s"),
    scratch_types=[
        pltpu.VMEM((_ROWS_PER_GROUP, _MAX_LEN), jnp.int32),
        pltpu.VMEM((_MAX_LEN, _ROWS_PER_GROUP), jnp.int32),
        pltpu.VMEM((_ROWS_PER_GROUP,), jnp.int32),
    ],
)()


def kernel(item_seq, item_seq_len):
    augmented = _sc_kernel(item_seq, item_seq_len, jnp.asarray(_PERM_G))
    return augmented, item_seq_len


# speedup vs baseline: 3.9689x; 3.9689x over previous
import functools

import jax
import jax.numpy as jnp
import numpy as np
from jax import lax
from jax.experimental import pallas as pl
from jax.experimental.pallas import tpu as pltpu
from jax.experimental.pallas import tpu_sc as plsc

_BATCH = 16384
_MAX_LEN = 200
_MASK_RATIO = 0.2

_NUM_WORKERS = 32
_ROWS_PER_GROUP = 16
_GROUPS = _BATCH // _ROWS_PER_GROUP
_GROUPS_PER_WORKER = _GROUPS // _NUM_WORKERS
_NUM_CORES = 2


def _threefry2x32_np(k0, k1, x0, x1):
    def rotl(x, d):
        return (x << np.uint32(d)) | (x >> np.uint32(32 - d))

    rotations = ((13, 15, 26, 6), (17, 29, 16, 24))
    ks = (np.uint32(k0), np.uint32(k1),
          np.uint32(k0) ^ np.uint32(k1) ^ np.uint32(0x1BD11BDA))
    x0 = (x0 + ks[0]).astype(np.uint32)
    x1 = (x1 + ks[1]).astype(np.uint32)
    for i in range(5):
        for r in rotations[i % 2]:
            x0 = (x0 + x1).astype(np.uint32)
            x1 = rotl(x1, r) ^ x0
        x0 = (x0 + ks[(i + 1) % 3]).astype(np.uint32)
        x1 = (x1 + ks[(i + 2) % 3] + np.uint32(i + 1)).astype(np.uint32)
    return x0, x1


def _score_perm_grouped() -> np.ndarray:
    size = _BATCH * _MAX_LEN
    idx = np.arange(size, dtype=np.uint64)
    hi = (idx >> np.uint64(32)).astype(np.uint32)
    lo = (idx & np.uint64(0xFFFFFFFF)).astype(np.uint32)
    o0, o1 = _threefry2x32_np(np.uint32(0), np.uint32(42), hi, lo)
    bits = o0 ^ o1
    fbits = (bits >> np.uint32(9)) | np.uint32(0x3F800000)
    scores = (fbits.view(np.float32) - np.float32(1.0)).reshape(_BATCH, _MAX_LEN)
    perm = np.argsort(scores, axis=1, kind="stable").astype(np.int32)
    return np.ascontiguousarray(
        perm.reshape(_GROUPS, _ROWS_PER_GROUP, _MAX_LEN).transpose(0, 2, 1))


_PERM_G = _score_perm_grouped()


_ROWS_PER_WORKER = _BATCH // _NUM_WORKERS
_NSLOT = 8


def _sc_body(item_hbm, len_hbm, perm_hbm, out_hbm,
             item_v, perm_v, len_v, in_item_sems, in_perm_sems, out_sems):
    wid = lax.axis_index("s") * _NUM_CORES + lax.axis_index("c")
    iota = lax.iota(jnp.int32, 16)
    zeros = jnp.zeros((16,), jnp.int32)
    base_g = wid * _GROUPS_PER_WORKER

    pltpu.sync_copy(len_hbm.at[pl.ds(wid * _ROWS_PER_WORKER, _ROWS_PER_WORKER)],
                    len_v)

    def start_in(g, b):
        gg = base_g + g
        pltpu.async_copy(item_hbm.at[pl.ds(gg * 16, 16), :], item_v.at[b],
                         in_item_sems.at[b])
        pltpu.async_copy(perm_hbm.at[gg], perm_v.at[b], in_perm_sems.at[b])

    def wait_in(g, b):
        gg = base_g + g
        pltpu.make_async_copy(item_hbm.at[pl.ds(gg * 16, 16), :], item_v.at[b],
                              in_item_sems.at[b]).wait()
        pltpu.make_async_copy(perm_hbm.at[gg], perm_v.at[b],
                              in_perm_sems.at[b]).wait()

    def start_out(g, b):
        gg = base_g + g
        pltpu.async_copy(item_v.at[b], out_hbm.at[pl.ds(gg * 16, 16), :],
                         out_sems.at[b])

    def wait_out(g, b):
        gg = base_g + g
        pltpu.make_async_copy(item_v.at[b], out_hbm.at[pl.ds(gg * 16, 16), :],
                              out_sems.at[b]).wait()

    for b in range(_NSLOT - 1):
        start_in(b, b)

    def outer(j, carry):
        for b in range(_NSLOT):
            g = j * _NSLOT + b
            wait_in(g, b)
            itv = item_v.at[b]
            pmv = perm_v.at[b]
            lens = len_v[pl.ds(g * 16, 16)]

            def p1(jj, nv):
                for u in range(4):
                    cj = zeros + (jj * 4 + u)
                    col = plsc.load_gather(itv, [iota, cj])
                    v = (cj < lens) & (col != 0)
                    nv = nv + v.astype(jnp.int32)
                return nv

            nv = lax.fori_loop(0, _MAX_LEN // 4, p1, zeros)
            m = (nv.astype(jnp.float32) * _MASK_RATIO).astype(jnp.int32)
            m = jnp.minimum(jnp.maximum(m, 1), nv)
            m = jnp.where((lens > 1) & (nv > 0), m, zeros)

            def cond(c):
                t, cnt = c
                return (t < _MAX_LEN) & (jnp.max(m - cnt) > 0)

            def body(c):
                t, cnt = c
                for u in range(8):
                    pcol = pmv[t + u]
                    itm = plsc.load_gather(itv, [iota, pcol])
                    v = (pcol < lens) & (itm != 0)
                    sel = v & (cnt < m)
                    plsc.store_scatter(itv, [iota, pcol], zeros, mask=sel)
                    cnt = cnt + v.astype(jnp.int32)
                return (t + 8, cnt)

            lax.while_loop(cond, body, (jnp.int32(0), zeros))
            start_out(g, b)

            pb = (b - 1) % _NSLOT

            @pl.when(g + _NSLOT - 1 < _GROUPS_PER_WORKER)
            def _():
                @pl.when(g >= 1)
                def _():
                    wait_out(g - 1, pb)
                start_in(g + _NSLOT - 1, pb)
        return carry

    lax.fori_loop(0, _GROUPS_PER_WORKER // _NSLOT, outer, 0)

    for b in range(_NSLOT):
        wait_out(_GROUPS_PER_WORKER - _NSLOT + b, b)


_sc_kernel = pl.kernel(
    _sc_body,
    out_type=jax.ShapeDtypeStruct((_BATCH, _MAX_LEN), jnp.int32),
    mesh=plsc.VectorSubcoreMesh(core_axis_name="c", subcore_axis_name="s"),
    compiler_params=pltpu.CompilerParams(use_tc_tiling_on_sc=False,
                                         needs_layout_passes=False),
    scratch_types=[
        pltpu.VMEM((_NSLOT, _ROWS_PER_GROUP, _MAX_LEN), jnp.int32),
        pltpu.VMEM((_NSLOT, _MAX_LEN, _ROWS_PER_GROUP), jnp.int32),
        pltpu.VMEM((_ROWS_PER_WORKER,), jnp.int32),
        pltpu.SemaphoreType.DMA((_NSLOT,)),
        pltpu.SemaphoreType.DMA((_NSLOT,)),
        pltpu.SemaphoreType.DMA((_NSLOT,)),
    ],
)


def kernel(item_seq, item_seq_len):
    augmented = _sc_kernel(item_seq, item_seq_len, jnp.asarray(_PERM_G))
    return augmented, item_seq_len
